# Initial kernel scaffold; baseline (speedup 1.0000x reference)
import jax
import jax.numpy as jnp
from jax.experimental import pallas as pl


def kernel(input_llr, check_index_tensor, var_index_tensor, w_ch, w_check, w_res):
    raise NotImplementedError("write your pallas kernel here")


# speedup vs baseline: 3.5678x; 3.5678x over previous
import functools

import jax
import jax.numpy as jnp
from jax import lax
from jax.experimental import pallas as pl
from jax.experimental.pallas import tpu as pltpu
from jax.experimental.pallas import tpu_sc as plsc

N = 8448
B = 128
K = 19
KP = K + 1
NC = 2
NS = 16
NW = NC * NS
NPT = N // NW
NCH = 6
CPT = NPT // NCH
IDXW = NCH * KP
LANES = 16
NG = B // LANES

_SIGN = -0x80000000
_MAG = 0x7FFFFFFF


def _iter_body(final, tbl, prv, wllr, llr, w3, idx, out,
               idx_v, rows_v, llr_v, prv_v, raw_v, out_v, w_v):
    wid = lax.axis_index("s") * NC + lax.axis_index("c")
    base = wid * NPT
    pltpu.sync_copy(idx.at[wid], idx_v)
    pltpu.sync_copy(w3.at[pl.ds(base * 3, NPT * 3 + LANES)], w_v)
    pltpu.sync_copy(wllr.at[pl.ds(base, NPT)], llr_v)
    pltpu.sync_copy(prv.at[pl.ds(base, NPT)], prv_v)

    def chunk_body(c, _):
        pltpu.sync_copy(tbl.at[idx_v.at[c]], rows_v)
        if final:
            pltpu.sync_copy(
                llr.at[pl.ds((base + c * NCH) * B, NCH * B)], raw_v)

        def node_body(i, _):
            j = c * NCH + i
            wv = w_v[pl.ds(j * 3, LANES)]
            wck = wv[0]
            wr0 = wv[1]
            wr1 = wv[2]
            r0 = i * KP
            for g in range(NG):
                sl = pl.ds(g * LANES, LANES)
                v = lax.bitcast_convert_type(rows_v[r0, sl], jnp.int32)
                sgn = v
                mag = v & _MAG
                for k in range(1, K):
                    v = lax.bitcast_convert_type(rows_v[r0 + k, sl],
                                                 jnp.int32)
                    sgn = sgn ^ v
                    mag = jnp.minimum(mag, v & _MAG)
                check = lax.bitcast_convert_type((sgn & _SIGN) | mag,
                                                 jnp.float32)
                own = rows_v[r0 + K, sl]
                res = (llr_v[j, sl] + wck * check
                       + wr0 * own + wr1 * prv_v[j, sl])
                if final:
                    res = 1.0 / (1.0 + jnp.exp(
                        -(res + raw_v[pl.ds(i * B + g * LANES, LANES)])))
                out_v[j, sl] = res
            return _

        lax.fori_loop(0, NCH, node_body, None)
        return _

    lax.fori_loop(0, CPT, chunk_body, None)
    pltpu.sync_copy(out_v, out.at[pl.ds(base, NPT)])


@functools.partial(jax.jit, static_argnames=("final",))
def _bp_iter(tbl, prv, wllr, llr, w3, idx, final):
    mesh = plsc.VectorSubcoreMesh(core_axis_name="c", subcore_axis_name="s")
    return pl.kernel(
        functools.partial(_iter_body, final),
        out_type=jax.ShapeDtypeStruct((N, B), jnp.float32),
        mesh=mesh,
        scratch_types=[
            pltpu.VMEM((CPT, IDXW), jnp.int32),
            pltpu.VMEM((IDXW, B), jnp.float32),
            pltpu.VMEM((NPT, B), jnp.float32),
            pltpu.VMEM((NPT, B), jnp.float32),
            pltpu.VMEM((NCH * B,), jnp.float32),
            pltpu.VMEM((NPT, B), jnp.float32),
            pltpu.VMEM((NPT * 3 + LANES,), jnp.float32),
        ],
    )(tbl, prv, wllr, llr, w3, idx)


def kernel(input_llr, check_index_tensor, var_index_tensor, w_ch, w_check,
           w_res):
    del var_index_tensor
    llr_flat = input_llr.T.reshape(-1)
    wllr_t = (input_llr * w_ch[None, :]).T
    own = jnp.arange(N, dtype=jnp.int32)[:, None]
    idx = jnp.concatenate(
        [check_index_tensor.astype(jnp.int32), own], axis=1)
    idx = idx.reshape(NW, CPT, IDXW)
    zeros = jnp.zeros_like(w_res[0])
    tbl = wllr_t
    prv = wllr_t
    for t in range(5):
        w3 = jnp.stack(
            [w_check, w_res[0], w_res[1] if t > 0 else zeros], axis=1)
        w3 = jnp.concatenate(
            [w3.reshape(-1), jnp.zeros((LANES,), jnp.float32)])
        new = _bp_iter(tbl, prv, wllr_t, llr_flat, w3, idx, final=(t == 4))
        prv, tbl = tbl, new
    return tbl.T


# speedup vs baseline: 5.6097x; 1.5723x over previous
import functools

import jax
import jax.numpy as jnp
from jax import lax
from jax.experimental import pallas as pl
from jax.experimental.pallas import tpu as pltpu
from jax.experimental.pallas import tpu_sc as plsc

N = 8448
B = 128
K = 19
KP = K + 1
NC = 2
NS = 16
NW = NC * NS
NPT = N // NW
NCH = 6
CPT = NPT // NCH
IDXW = NCH * KP
LANES = 16
NG = B // LANES

_SIGN = -0x80000000
_MAG = 0x7FFFFFFF


def _iter_body(final, tbl, prv, wllr, llr, w3, idx, out,
               idx_v, rows0_v, rows1_v, io_v, prv_v, raw_v, w_v,
               sem0, sem1):
    wid = lax.axis_index("s") * NC + lax.axis_index("c")
    base = wid * NPT
    pltpu.sync_copy(idx.at[wid], idx_v)
    pltpu.sync_copy(w3.at[pl.ds(base * 3, NPT * 3 + LANES)], w_v)
    pltpu.sync_copy(wllr.at[pl.ds(base, NPT)], io_v)
    pltpu.sync_copy(prv.at[pl.ds(base, NPT)], prv_v)

    bufs = ((rows0_v, sem0), (rows1_v, sem1))

    def issue(c, b):
        pltpu.async_copy(tbl.at[idx_v.at[c]], bufs[b][0], bufs[b][1])

    def compute(c, rows_v):
        if final:
            pltpu.sync_copy(
                llr.at[pl.ds((base + c * NCH) * B, NCH * B)], raw_v)

        def node_body(i, _):
            j = c * NCH + i
            wv = w_v[pl.ds(j * 3, LANES)]
            wck = wv[0]
            wr0 = wv[1]
            wr1 = wv[2]
            r0 = i * KP
            for g in range(NG):
                sl = pl.ds(g * LANES, LANES)
                v = lax.bitcast_convert_type(rows_v[r0, sl], jnp.int32)
                sgn = v
                mag = v & _MAG
                for k in range(1, K):
                    v = lax.bitcast_convert_type(rows_v[r0 + k, sl],
                                                 jnp.int32)
                    sgn = sgn ^ v
                    mag = jnp.minimum(mag, v & _MAG)
                check = lax.bitcast_convert_type((sgn & _SIGN) | mag,
                                                 jnp.float32)
                own = rows_v[r0 + K, sl]
                res = (io_v[j, sl] + wck * check
                       + wr0 * own + wr1 * prv_v[j, sl])
                if final:
                    res = 1.0 / (1.0 + jnp.exp(
                        -(res + raw_v[pl.ds(i * B + g * LANES, LANES)])))
                io_v[j, sl] = res
            return _

        lax.fori_loop(0, NCH, node_body, None)

    issue(0, 0)
    issue(1, 1)

    def outer(c0, _):
        for b in range(2):
            c = c0 + b
            pltpu.make_async_copy(
                tbl.at[idx_v.at[c]], bufs[b][0], bufs[b][1]).wait()
            compute(c, bufs[b][0])

            @pl.when(c + 2 < CPT)
            def _issue_next():
                issue(c + 2, b)
        return _

    lax.fori_loop(0, CPT // 2, lambda s, x: outer(s * 2, x), None)
    pltpu.sync_copy(io_v, out.at[pl.ds(base, NPT)])


@functools.partial(jax.jit, static_argnames=("final",))
def _bp_iter(tbl, prv, wllr, llr, w3, idx, final):
    mesh = plsc.VectorSubcoreMesh(core_axis_name="c", subcore_axis_name="s")
    return pl.kernel(
        functools.partial(_iter_body, final),
        out_type=jax.ShapeDtypeStruct((N, B), jnp.float32),
        mesh=mesh,
        scratch_types=[
            pltpu.VMEM((CPT, IDXW), jnp.int32),
            pltpu.VMEM((IDXW, B), jnp.float32),
            pltpu.VMEM((IDXW, B), jnp.float32),
            pltpu.VMEM((NPT, B), jnp.float32),
            pltpu.VMEM((NPT, B), jnp.float32),
            pltpu.VMEM((NCH * B,), jnp.float32),
            pltpu.VMEM((NPT * 3 + LANES,), jnp.float32),
            pltpu.SemaphoreType.DMA,
            pltpu.SemaphoreType.DMA,
        ],
    )(tbl, prv, wllr, llr, w3, idx)


def kernel(input_llr, check_index_tensor, var_index_tensor, w_ch, w_check,
           w_res):
    del var_index_tensor
    llr_flat = input_llr.T.reshape(-1)
    wllr_t = (input_llr * w_ch[None, :]).T
    own = jnp.arange(N, dtype=jnp.int32)[:, None]
    idx = jnp.concatenate(
        [check_index_tensor.astype(jnp.int32), own], axis=1)
    idx = idx.reshape(NW, CPT, IDXW)
    zeros = jnp.zeros_like(w_res[0])
    tbl = wllr_t
    prv = wllr_t
    for t in range(5):
        w3 = jnp.stack(
            [w_check, w_res[0], w_res[1] if t > 0 else zeros], axis=1)
        w3 = jnp.concatenate(
            [w3.reshape(-1), jnp.zeros((LANES,), jnp.float32)])
        new = _bp_iter(tbl, prv, wllr_t, llr_flat, w3, idx, final=(t == 4))
        prv, tbl = tbl, new
    return tbl.T


# speedup vs baseline: 6.0874x; 1.0851x over previous
import functools

import jax
import jax.numpy as jnp
from jax import lax
from jax.experimental import pallas as pl
from jax.experimental.pallas import tpu as pltpu
from jax.experimental.pallas import tpu_sc as plsc

N = 8448
B = 128
K = 19
KP = K + 1
NC = 2
NS = 16
NW = NC * NS
NPT = N // NW
NCH = 6
CPT = NPT // NCH
IDXW = NCH * KP
LANES = 16
NG = B // LANES

_SIGN = -0x80000000
_MAG = 0x7FFFFFFF


def _iter_body(final, tbl, prv, wllr, w3, idx, out,
               idx_v, rows0_v, rows1_v, io_v, prv_v, w_v,
               sem0, sem1):
    wid = lax.axis_index("s") * NC + lax.axis_index("c")
    base = wid * NPT
    pltpu.sync_copy(idx.at[wid], idx_v)

    bufs = ((rows0_v, sem0), (rows1_v, sem1))

    def issue(c, b):
        pltpu.async_copy(tbl.at[idx_v.at[c]], bufs[b][0], bufs[b][1])

    issue(0, 0)
    issue(1, 1)
    pltpu.sync_copy(w3.at[pl.ds(base * 3, NPT * 3 + LANES)], w_v)
    pltpu.sync_copy(wllr.at[pl.ds(base, NPT)], io_v)
    pltpu.sync_copy(prv.at[pl.ds(base, NPT)], prv_v)

    def compute(c, rows_v):
        def node_body(i, _):
            j = c * NCH + i
            wv = w_v[pl.ds(j * 3, LANES)]
            wck = wv[0]
            wr0 = wv[1]
            wr1 = wv[2]
            r0 = i * KP
            for g in range(NG):
                sl = pl.ds(g * LANES, LANES)
                v = lax.bitcast_convert_type(rows_v[r0, sl], jnp.int32)
                sgn = v
                mag = v & _MAG
                for k in range(1, K):
                    v = lax.bitcast_convert_type(rows_v[r0 + k, sl],
                                                 jnp.int32)
                    sgn = sgn ^ v
                    mag = jnp.minimum(mag, v & _MAG)
                check = lax.bitcast_convert_type((sgn & _SIGN) | mag,
                                                 jnp.float32)
                own = rows_v[r0 + K, sl]
                res = (io_v[j, sl] + wck * check
                       + wr0 * own + wr1 * prv_v[j, sl])
                if final:
                    res = 1.0 / (1.0 + jnp.exp(-res))
                io_v[j, sl] = res
            return _

        lax.fori_loop(0, NCH, node_body, None)

    def outer(c0, _):
        for b in range(2):
            c = c0 + b
            pltpu.make_async_copy(
                tbl.at[idx_v.at[c]], bufs[b][0], bufs[b][1]).wait()
            compute(c, bufs[b][0])

            @pl.when(c + 2 < CPT)
            def _issue_next():
                issue(c + 2, b)
        return _

    lax.fori_loop(0, CPT // 2, lambda s, x: outer(s * 2, x), None)
    pltpu.sync_copy(io_v, out.at[pl.ds(base, NPT)])


@functools.partial(jax.jit, static_argnames=("final",))
def _bp_iter(tbl, prv, wllr, w3, idx, final):
    mesh = plsc.VectorSubcoreMesh(core_axis_name="c", subcore_axis_name="s")
    return pl.kernel(
        functools.partial(_iter_body, final),
        out_type=jax.ShapeDtypeStruct((N, B), jnp.float32),
        mesh=mesh,
        scratch_types=[
            pltpu.VMEM((CPT, IDXW), jnp.int32),
            pltpu.VMEM((IDXW, B), jnp.float32),
            pltpu.VMEM((IDXW, B), jnp.float32),
            pltpu.VMEM((NPT, B), jnp.float32),
            pltpu.VMEM((NPT, B), jnp.float32),
            pltpu.VMEM((NPT * 3 + LANES,), jnp.float32),
            pltpu.SemaphoreType.DMA,
            pltpu.SemaphoreType.DMA,
        ],
    )(tbl, prv, wllr, w3, idx)


def kernel(input_llr, check_index_tensor, var_index_tensor, w_ch, w_check,
           w_res):
    del var_index_tensor
    llr_t = input_llr.T
    wllr_t = (input_llr * w_ch[None, :]).T
    own = jnp.arange(N, dtype=jnp.int32)[:, None]
    idx = jnp.concatenate(
        [check_index_tensor.astype(jnp.int32), own], axis=1)
    idx = idx.reshape(NW, CPT, IDXW)
    zeros = jnp.zeros_like(w_res[0])
    tbl = wllr_t
    prv = wllr_t
    for t in range(5):
        w3 = jnp.stack(
            [w_check, w_res[0], w_res[1] if t > 0 else zeros], axis=1)
        w3 = jnp.concatenate(
            [w3.reshape(-1), jnp.zeros((LANES,), jnp.float32)])
        wl = wllr_t + llr_t if t == 4 else wllr_t
        new = _bp_iter(tbl, prv, wl, w3, idx, final=(t == 4))
        prv, tbl = tbl, new
    return tbl.T
